# baseline (speedup 1.0000x reference)
import jax
import jax.numpy as jnp
import numpy as np
from jax.experimental import pallas as pl
from jax.experimental.pallas import tpu as pltpu

B, S, T = 16, 512, 20
VOCAB, E, H = 262, 64, 64
VP = 264
VP2 = 2 * VP
NW = 2048
NR = NW // 2
N = B * S


def _lstm_kernel(cp_ref, emb2_ref, wf_ref, wr_ref, bias_ref,
                 ones_ref, hc0_ref, out_ref, x_ref, outf_ref):
    f32 = jnp.float32
    bf16 = jnp.bfloat16
    lane2 = jax.lax.broadcasted_iota(jnp.int32, (NR, VP2), 1)

    def sg(z):
        return 0.5 * jnp.tanh(z) + 0.5

    for t in range(T):
        ia = cp_ref[:, t:t + 1]
        ib = cp_ref[:, T + t:T + t + 1]
        tgt = jnp.where(lane2 < VP, ia, ib + VP)
        onehot = (tgt == lane2).astype(bf16)
        x_ref[t] = jnp.dot(onehot, emb2_ref[...],
                           preferred_element_type=f32).astype(bf16)

    la = cp_ref[:, 2 * T:2 * T + 1]
    lb = cp_ref[:, 2 * T + 1:2 * T + 2]
    lane128 = jax.lax.broadcasted_iota(jnp.int32, (NR, 2 * H), 1)
    len_mat = jnp.where(lane128 < H, la, lb)
    bias_f = bias_ref[0, :][None, :]
    bias_r = bias_ref[1, :][None, :]

    def step(h, c, x, bias, w_ref, m, mask_state):
        inp = jnp.concatenate([x, h.astype(bf16)], axis=1)
        gates = jnp.dot(inp, w_ref[...], preferred_element_type=f32) + bias
        i = sg(gates[:, 0:2*H])
        f = sg(gates[:, 2*H:4*H])
        g = jnp.tanh(gates[:, 4*H:6*H])
        o = sg(gates[:, 6*H:8*H])
        c_new = f * c + i * g
        h_new = o * jnp.tanh(c_new)
        if mask_state:
            h = jnp.where(m, h_new, h)
            c = jnp.where(m, c_new, c)
        else:
            h, c = h_new, c_new
        return h, c, jnp.where(m, h_new, 0.0)

    h = jnp.broadcast_to(hc0_ref[0, :][None, :], (NR, 2 * H))
    c = jnp.broadcast_to(hc0_ref[1, :][None, :], (NR, 2 * H))
    for t in range(T):
        m = len_mat > t
        h, c, out = step(h, c, x_ref[t], bias_f, wf_ref, m, False)
        outf_ref[t] = out

    h = jnp.broadcast_to(hc0_ref[2, :][None, :], (NR, 2 * H))
    c = jnp.broadcast_to(hc0_ref[3, :][None, :], (NR, 2 * H))
    acc_f = jnp.zeros((NR, 2 * H), f32)
    acc_r = jnp.zeros((NR, 2 * H), f32)
    for t in range(T - 1, -1, -1):
        m = len_mat > t
        h, c, out_r = step(h, c, x_ref[t], bias_r, wr_ref, m, True)
        out_f = outf_ref[t]
        obf = jnp.concatenate([out_f, out_r], axis=1).astype(bf16)
        logit = jnp.dot(obf, ones_ref[...], preferred_element_type=f32)
        att = sg(logit)
        acc_f = acc_f + att * out_f
        acc_r = acc_r + att * out_r
    out_ref[0:NR, 0:H] = acc_f[:, 0:H]
    out_ref[0:NR, H:2*H] = acc_r[:, 0:H]
    out_ref[NR:NW, 0:H] = acc_f[:, H:2*H]
    out_ref[NR:NW, H:2*H] = acc_r[:, H:2*H]


@jax.jit
def _run(colpack, emb2, wf, wr, bias, ones_bd, hc0):
    grid = (N // NW,)
    out = pl.pallas_call(
        _lstm_kernel,
        grid=grid,
        in_specs=[
            pl.BlockSpec((NR, 64), lambda i: (i, 0)),
            pl.BlockSpec((VP2, 2 * H), lambda i: (0, 0)),
            pl.BlockSpec((4 * H, 8 * H), lambda i: (0, 0)),
            pl.BlockSpec((4 * H, 8 * H), lambda i: (0, 0)),
            pl.BlockSpec((2, 8 * H), lambda i: (0, 0)),
            pl.BlockSpec((4 * H, 2 * H), lambda i: (0, 0)),
            pl.BlockSpec((4, 2 * H), lambda i: (0, 0)),
        ],
        out_specs=pl.BlockSpec((NW, 2 * H), lambda i: (i, 0)),
        out_shape=jax.ShapeDtypeStruct((N, 2 * H), jnp.float32),
        scratch_shapes=[
            pltpu.VMEM((T, NR, 2 * H), jnp.bfloat16),
            pltpu.VMEM((T, NR, 2 * H), jnp.float32),
        ],
    )(colpack, emb2, wf, wr, bias, ones_bd, hc0)
    return out.reshape(B, S, 2 * H)


def _pair_cols(w):
    K = w.shape[0]
    out = jnp.zeros((2 * K, 8 * H), w.dtype)
    for q in range(4):
        blk = w[:, q * H:(q + 1) * H]
        out = out.at[0:E, 2 * q * H:(2 * q + 1) * H].set(blk[0:E])
        out = out.at[2 * E:2 * E + H, 2 * q * H:(2 * q + 1) * H].set(blk[E:])
        out = out.at[E:2 * E, (2 * q + 1) * H:(2 * q + 2) * H].set(blk[0:E])
        out = out.at[2 * E + H:, (2 * q + 1) * H:(2 * q + 2) * H].set(blk[E:])
    return out


def _pair_gates_vec(b):
    return jnp.concatenate([jnp.concatenate([b[q * H:(q + 1) * H]] * 2)
                            for q in range(4)])


def kernel(char_indices, token_lengths, emb_table, Wih_f, Whh_f, bih_f, bhh_f,
           Wih_r, Whh_r, bih_r, bhh_r, h0, c0, attn_w):
    idx = char_indices.reshape(N, T).astype(jnp.int32)
    lengths = token_lengths.reshape(N).astype(jnp.int32)
    G = N // NW
    idx_g = idx.reshape(G, 2, NR, T)
    len_g = lengths.reshape(G, 2, NR, 1)
    colpack = jnp.concatenate(
        [idx_g[:, 0], idx_g[:, 1], len_g[:, 0], len_g[:, 1],
         jnp.zeros((G, NR, 64 - 2 * T - 2), jnp.int32)], axis=-1)
    colpack = colpack.reshape(G * NR, 64)

    emb_pad = jnp.zeros((VP, E), jnp.float32).at[:VOCAB].set(emb_table)
    emb2 = jnp.zeros((VP2, 2 * H), jnp.float32)
    emb2 = emb2.at[0:VP, 0:E].set(emb_pad).at[VP:, E:].set(emb_pad)
    emb2 = emb2.astype(jnp.bfloat16)

    gate_scale = jnp.concatenate(
        [jnp.full((H,), 0.5), jnp.full((H,), 0.5),
         jnp.ones((H,)), jnp.full((H,), 0.5)]).astype(jnp.float32)
    wcat_f = jnp.concatenate([Wih_f.T, Whh_f.T], axis=0) * gate_scale[None, :]
    wcat_r = jnp.concatenate([Wih_r.T, Whh_r.T], axis=0) * gate_scale[None, :]
    wf = _pair_cols(wcat_f).astype(jnp.bfloat16)
    wr = _pair_cols(wcat_r).astype(jnp.bfloat16)
    bias = jnp.stack([_pair_gates_vec((bih_f + bhh_f) * gate_scale),
                      _pair_gates_vec((bih_r + bhh_r) * gate_scale)])
    attn = 0.5 * jnp.stack([jnp.concatenate([attn_w[0, 0:H]] * 2),
                            jnp.concatenate([attn_w[0, H:]] * 2)])
    lane = np.arange(2 * H)
    bd = jnp.asarray((lane[:, None] // H) == (lane[None, :] // H),
                     jnp.float32)
    ones_bd = jnp.concatenate([attn[0][:, None] * bd,
                               attn[1][:, None] * bd]).astype(jnp.bfloat16)
    hc0 = jnp.stack([jnp.concatenate([h0[0, 0]] * 2),
                     jnp.concatenate([c0[0, 0]] * 2),
                     jnp.concatenate([h0[1, 0]] * 2),
                     jnp.concatenate([c0[1, 0]] * 2)])
    return _run(colpack, emb2, wf, wr, bias, ones_bd, hc0)


# speedup vs baseline: 1.0032x; 1.0032x over previous
import jax
import jax.numpy as jnp
import numpy as np
from jax.experimental import pallas as pl
from jax.experimental.pallas import tpu as pltpu

B, S, T = 16, 512, 20
VOCAB, E, H = 262, 64, 64
VP = 264
VP2 = 2 * VP
NW = 1024
NR = NW // 2
N = B * S


def _lstm_kernel(cp_ref, emb2_ref, wf_ref, wr_ref, bias_ref,
                 ones_ref, hc0_ref, out_ref, x_ref, outf_ref):
    f32 = jnp.float32
    bf16 = jnp.bfloat16
    lane2 = jax.lax.broadcasted_iota(jnp.int16, (NR, VP2), 1)

    def sg(z):
        return 0.5 * jnp.tanh(z) + 0.5

    for t in range(T):
        ia = cp_ref[:, t:t + 1].astype(jnp.int16)
        ib = cp_ref[:, T + t:T + t + 1].astype(jnp.int16)
        tgt = jnp.where(lane2 < VP, ia, ib + jnp.int16(VP))
        onehot = (tgt == lane2).astype(bf16)
        x_ref[t] = jnp.dot(onehot, emb2_ref[...],
                           preferred_element_type=f32).astype(bf16)

    la = cp_ref[:, 2 * T:2 * T + 1]
    lb = cp_ref[:, 2 * T + 1:2 * T + 2]
    lane128 = jax.lax.broadcasted_iota(jnp.int32, (NR, 2 * H), 1)
    len_mat = jnp.where(lane128 < H, la, lb)
    bias_f = bias_ref[0, :][None, :]
    bias_r = bias_ref[1, :][None, :]

    def step(h, c, x, bias, w_ref, m, mask_state):
        inp = jnp.concatenate([x, h.astype(bf16)], axis=1)
        gates = jnp.dot(inp, w_ref[...], preferred_element_type=f32) + bias
        i = sg(gates[:, 0:2*H])
        f = sg(gates[:, 2*H:4*H])
        g = jnp.tanh(gates[:, 4*H:6*H])
        o = sg(gates[:, 6*H:8*H])
        c_new = f * c + i * g
        h_new = o * jnp.tanh(c_new)
        if mask_state:
            h = jnp.where(m, h_new, h)
            c = jnp.where(m, c_new, c)
        else:
            h, c = h_new, c_new
        return h, c, jnp.where(m, h_new, 0.0)

    h = jnp.broadcast_to(hc0_ref[0, :][None, :], (NR, 2 * H))
    c = jnp.broadcast_to(hc0_ref[1, :][None, :], (NR, 2 * H))
    for t in range(T):
        m = len_mat > t
        h, c, out = step(h, c, x_ref[t], bias_f, wf_ref, m, False)
        outf_ref[t] = out

    h = jnp.broadcast_to(hc0_ref[2, :][None, :], (NR, 2 * H))
    c = jnp.broadcast_to(hc0_ref[3, :][None, :], (NR, 2 * H))
    acc_f = jnp.zeros((NR, 2 * H), f32)
    acc_r = jnp.zeros((NR, 2 * H), f32)
    for t in range(T - 1, -1, -1):
        m = len_mat > t
        h, c, out_r = step(h, c, x_ref[t], bias_r, wr_ref, m, True)
        out_f = outf_ref[t]
        obf = jnp.concatenate([out_f, out_r], axis=1).astype(bf16)
        logit = jnp.dot(obf, ones_ref[...], preferred_element_type=f32)
        att = sg(logit)
        acc_f = acc_f + att * out_f
        acc_r = acc_r + att * out_r
    out_ref[0:NR, 0:H] = acc_f[:, 0:H]
    out_ref[0:NR, H:2*H] = acc_r[:, 0:H]
    out_ref[NR:NW, 0:H] = acc_f[:, H:2*H]
    out_ref[NR:NW, H:2*H] = acc_r[:, H:2*H]


@jax.jit
def _run(colpack, emb2, wf, wr, bias, ones_bd, hc0):
    grid = (N // NW,)
    out = pl.pallas_call(
        _lstm_kernel,
        grid=grid,
        in_specs=[
            pl.BlockSpec((NR, 64), lambda i: (i, 0)),
            pl.BlockSpec((VP2, 2 * H), lambda i: (0, 0)),
            pl.BlockSpec((4 * H, 8 * H), lambda i: (0, 0)),
            pl.BlockSpec((4 * H, 8 * H), lambda i: (0, 0)),
            pl.BlockSpec((2, 8 * H), lambda i: (0, 0)),
            pl.BlockSpec((4 * H, 2 * H), lambda i: (0, 0)),
            pl.BlockSpec((4, 2 * H), lambda i: (0, 0)),
        ],
        out_specs=pl.BlockSpec((NW, 2 * H), lambda i: (i, 0)),
        out_shape=jax.ShapeDtypeStruct((N, 2 * H), jnp.float32),
        scratch_shapes=[
            pltpu.VMEM((T, NR, 2 * H), jnp.bfloat16),
            pltpu.VMEM((T, NR, 2 * H), jnp.float32),
        ],
    )(colpack, emb2, wf, wr, bias, ones_bd, hc0)
    return out.reshape(B, S, 2 * H)


def _pair_cols(w):
    K = w.shape[0]
    out = jnp.zeros((2 * K, 8 * H), w.dtype)
    for q in range(4):
        blk = w[:, q * H:(q + 1) * H]
        out = out.at[0:E, 2 * q * H:(2 * q + 1) * H].set(blk[0:E])
        out = out.at[2 * E:2 * E + H, 2 * q * H:(2 * q + 1) * H].set(blk[E:])
        out = out.at[E:2 * E, (2 * q + 1) * H:(2 * q + 2) * H].set(blk[0:E])
        out = out.at[2 * E + H:, (2 * q + 1) * H:(2 * q + 2) * H].set(blk[E:])
    return out


def _pair_gates_vec(b):
    return jnp.concatenate([jnp.concatenate([b[q * H:(q + 1) * H]] * 2)
                            for q in range(4)])


def kernel(char_indices, token_lengths, emb_table, Wih_f, Whh_f, bih_f, bhh_f,
           Wih_r, Whh_r, bih_r, bhh_r, h0, c0, attn_w):
    idx = char_indices.reshape(N, T).astype(jnp.int32)
    lengths = token_lengths.reshape(N).astype(jnp.int32)
    G = N // NW
    idx_g = idx.reshape(G, 2, NR, T)
    len_g = lengths.reshape(G, 2, NR, 1)
    colpack = jnp.concatenate(
        [idx_g[:, 0], idx_g[:, 1], len_g[:, 0], len_g[:, 1],
         jnp.zeros((G, NR, 64 - 2 * T - 2), jnp.int32)], axis=-1)
    colpack = colpack.reshape(G * NR, 64)

    emb_pad = jnp.zeros((VP, E), jnp.float32).at[:VOCAB].set(emb_table)
    emb2 = jnp.zeros((VP2, 2 * H), jnp.float32)
    emb2 = emb2.at[0:VP, 0:E].set(emb_pad).at[VP:, E:].set(emb_pad)
    emb2 = emb2.astype(jnp.bfloat16)

    gate_scale = jnp.concatenate(
        [jnp.full((H,), 0.5), jnp.full((H,), 0.5),
         jnp.ones((H,)), jnp.full((H,), 0.5)]).astype(jnp.float32)
    wcat_f = jnp.concatenate([Wih_f.T, Whh_f.T], axis=0) * gate_scale[None, :]
    wcat_r = jnp.concatenate([Wih_r.T, Whh_r.T], axis=0) * gate_scale[None, :]
    wf = _pair_cols(wcat_f).astype(jnp.bfloat16)
    wr = _pair_cols(wcat_r).astype(jnp.bfloat16)
    bias = jnp.stack([_pair_gates_vec((bih_f + bhh_f) * gate_scale),
                      _pair_gates_vec((bih_r + bhh_r) * gate_scale)])
    attn = 0.5 * jnp.stack([jnp.concatenate([attn_w[0, 0:H]] * 2),
                            jnp.concatenate([attn_w[0, H:]] * 2)])
    lane = np.arange(2 * H)
    bd = jnp.asarray((lane[:, None] // H) == (lane[None, :] // H),
                     jnp.float32)
    ones_bd = jnp.concatenate([attn[0][:, None] * bd,
                               attn[1][:, None] * bd]).astype(jnp.bfloat16)
    hc0 = jnp.stack([jnp.concatenate([h0[0, 0]] * 2),
                     jnp.concatenate([c0[0, 0]] * 2),
                     jnp.concatenate([h0[1, 0]] * 2),
                     jnp.concatenate([c0[1, 0]] * 2)])
    return _run(colpack, emb2, wf, wr, bias, ones_bd, hc0)


# speedup vs baseline: 1.0772x; 1.0738x over previous
import jax
import jax.numpy as jnp
import numpy as np
from jax.experimental import pallas as pl
from jax.experimental.pallas import tpu as pltpu

B, S, T = 16, 512, 20
VOCAB, E, H = 262, 64, 64
VP = 264
VP2 = 2 * VP
NW = 1024
NR = NW // 2
N = B * S


def _lstm_kernel(cp_ref, emb2_ref, wf_ref, wr_ref, bias_ref,
                 ones_ref, hc0_ref, out_ref, x_ref, outf_ref):
    f32 = jnp.float32
    bf16 = jnp.bfloat16
    lane2 = jax.lax.broadcasted_iota(jnp.int32, (NR, VP2), 1)

    def sg(z):
        return 0.5 * jnp.tanh(z) + 0.5

    for t in range(T):
        ia = cp_ref[:, t:t + 1]
        ib = cp_ref[:, T + t:T + t + 1]
        tgt = jnp.where(lane2 < VP, ia, ib + VP)
        onehot = (tgt == lane2).astype(bf16)
        x_ref[t] = jnp.dot(onehot, emb2_ref[...],
                           preferred_element_type=f32).astype(bf16)

    la = cp_ref[:, 2 * T:2 * T + 1]
    lb = cp_ref[:, 2 * T + 1:2 * T + 2]
    lane128 = jax.lax.broadcasted_iota(jnp.int32, (NR, 2 * H), 1)
    len_mat = jnp.where(lane128 < H, la, lb)
    bias_f = bias_ref[0, :][None, :]
    bias_r = bias_ref[1, :][None, :]

    def step(h, c, x, bias, w_ref, m, mask_state):
        inp = jnp.concatenate([x, h.astype(bf16)], axis=1)
        gates = jnp.dot(inp, w_ref[...], preferred_element_type=f32) + bias
        i = sg(gates[:, 0:2*H])
        f = sg(gates[:, 2*H:4*H])
        g = jnp.tanh(gates[:, 4*H:6*H])
        o = sg(gates[:, 6*H:8*H])
        c_new = f * c + i * g
        h_new = o * jnp.tanh(c_new)
        if mask_state:
            h = jnp.where(m, h_new, h)
            c = jnp.where(m, c_new, c)
        else:
            h, c = h_new, c_new
        return h, c, jnp.where(m, h_new, 0.0)

    h = jnp.broadcast_to(hc0_ref[0, :][None, :], (NR, 2 * H))
    c = jnp.broadcast_to(hc0_ref[1, :][None, :], (NR, 2 * H))
    for t in range(T):
        m = len_mat > t
        h, c, out = step(h, c, x_ref[t], bias_f, wf_ref, m, False)
        outf_ref[t] = out

    h = jnp.broadcast_to(hc0_ref[2, :][None, :], (NR, 2 * H))
    c = jnp.broadcast_to(hc0_ref[3, :][None, :], (NR, 2 * H))
    acc_f = jnp.zeros((NR, 2 * H), f32)
    acc_r = jnp.zeros((NR, 2 * H), f32)
    for t in range(T - 1, -1, -1):
        m = len_mat > t
        h, c, out_r = step(h, c, x_ref[t], bias_r, wr_ref, m, True)
        out_f = outf_ref[t]
        obf = jnp.concatenate([out_f, out_r], axis=1).astype(bf16)
        logit = jnp.dot(obf, ones_ref[...], preferred_element_type=f32)
        att = sg(logit)
        acc_f = acc_f + att * out_f
        acc_r = acc_r + att * out_r
    out_ref[0:NR, 0:H] = acc_f[:, 0:H]
    out_ref[0:NR, H:2*H] = acc_r[:, 0:H]
    out_ref[NR:NW, 0:H] = acc_f[:, H:2*H]
    out_ref[NR:NW, H:2*H] = acc_r[:, H:2*H]


@jax.jit
def _run(colpack, emb2, wf, wr, bias, ones_bd, hc0):
    grid = (N // NW,)
    out = pl.pallas_call(
        _lstm_kernel,
        grid=grid,
        in_specs=[
            pl.BlockSpec((NR, 64), lambda i: (i, 0)),
            pl.BlockSpec((VP2, 2 * H), lambda i: (0, 0)),
            pl.BlockSpec((4 * H, 8 * H), lambda i: (0, 0)),
            pl.BlockSpec((4 * H, 8 * H), lambda i: (0, 0)),
            pl.BlockSpec((2, 8 * H), lambda i: (0, 0)),
            pl.BlockSpec((4 * H, 2 * H), lambda i: (0, 0)),
            pl.BlockSpec((4, 2 * H), lambda i: (0, 0)),
        ],
        out_specs=pl.BlockSpec((NW, 2 * H), lambda i: (i, 0)),
        out_shape=jax.ShapeDtypeStruct((N, 2 * H), jnp.float32),
        scratch_shapes=[
            pltpu.VMEM((T, NR, 2 * H), jnp.bfloat16),
            pltpu.VMEM((T, NR, 2 * H), jnp.float32),
        ],
    )(colpack, emb2, wf, wr, bias, ones_bd, hc0)
    return out.reshape(B, S, 2 * H)


def _pair_cols(w):
    K = w.shape[0]
    out = jnp.zeros((2 * K, 8 * H), w.dtype)
    for q in range(4):
        blk = w[:, q * H:(q + 1) * H]
        out = out.at[0:E, 2 * q * H:(2 * q + 1) * H].set(blk[0:E])
        out = out.at[2 * E:2 * E + H, 2 * q * H:(2 * q + 1) * H].set(blk[E:])
        out = out.at[E:2 * E, (2 * q + 1) * H:(2 * q + 2) * H].set(blk[0:E])
        out = out.at[2 * E + H:, (2 * q + 1) * H:(2 * q + 2) * H].set(blk[E:])
    return out


def _pair_gates_vec(b):
    return jnp.concatenate([jnp.concatenate([b[q * H:(q + 1) * H]] * 2)
                            for q in range(4)])


def kernel(char_indices, token_lengths, emb_table, Wih_f, Whh_f, bih_f, bhh_f,
           Wih_r, Whh_r, bih_r, bhh_r, h0, c0, attn_w):
    idx = char_indices.reshape(N, T).astype(jnp.int32)
    lengths = token_lengths.reshape(N).astype(jnp.int32)
    G = N // NW
    idx_g = idx.reshape(G, 2, NR, T)
    len_g = lengths.reshape(G, 2, NR, 1)
    colpack = jnp.concatenate(
        [idx_g[:, 0], idx_g[:, 1], len_g[:, 0], len_g[:, 1],
         jnp.zeros((G, NR, 64 - 2 * T - 2), jnp.int32)], axis=-1)
    colpack = colpack.reshape(G * NR, 64)

    emb_pad = jnp.zeros((VP, E), jnp.float32).at[:VOCAB].set(emb_table)
    emb2 = jnp.zeros((VP2, 2 * H), jnp.float32)
    emb2 = emb2.at[0:VP, 0:E].set(emb_pad).at[VP:, E:].set(emb_pad)
    emb2 = emb2.astype(jnp.bfloat16)

    gate_scale = jnp.concatenate(
        [jnp.full((H,), 0.5), jnp.full((H,), 0.5),
         jnp.ones((H,)), jnp.full((H,), 0.5)]).astype(jnp.float32)
    wcat_f = jnp.concatenate([Wih_f.T, Whh_f.T], axis=0) * gate_scale[None, :]
    wcat_r = jnp.concatenate([Wih_r.T, Whh_r.T], axis=0) * gate_scale[None, :]
    wf = _pair_cols(wcat_f).astype(jnp.bfloat16)
    wr = _pair_cols(wcat_r).astype(jnp.bfloat16)
    bias = jnp.stack([_pair_gates_vec((bih_f + bhh_f) * gate_scale),
                      _pair_gates_vec((bih_r + bhh_r) * gate_scale)])
    attn = 0.5 * jnp.stack([jnp.concatenate([attn_w[0, 0:H]] * 2),
                            jnp.concatenate([attn_w[0, H:]] * 2)])
    lane = np.arange(2 * H)
    bd = jnp.asarray((lane[:, None] // H) == (lane[None, :] // H),
                     jnp.float32)
    ones_bd = jnp.concatenate([attn[0][:, None] * bd,
                               attn[1][:, None] * bd]).astype(jnp.bfloat16)
    hc0 = jnp.stack([jnp.concatenate([h0[0, 0]] * 2),
                     jnp.concatenate([c0[0, 0]] * 2),
                     jnp.concatenate([h0[1, 0]] * 2),
                     jnp.concatenate([c0[1, 0]] * 2)])
    return _run(colpack, emb2, wf, wr, bias, ones_bd, hc0)
